# Initial kernel scaffold; baseline (speedup 1.0000x reference)
import jax
import jax.numpy as jnp
from jax.experimental import pallas as pl


def kernel(x, dim, group_indices, temperature):
    raise NotImplementedError("write your pallas kernel here")


# speedup vs baseline: 2.4174x; 2.4174x over previous
import functools

import jax
import jax.numpy as jnp
from jax import lax
from jax.experimental import pallas as pl
from jax.experimental.pallas import tpu as pltpu
from jax.experimental.pallas import tpu_sc as plsc

_N = 320000
_D = 128
_S = 10000

_NSUB = 16
_ROWS_PER_SUB = _N // _NSUB
_R = 80
_TCH = _ROWS_PER_SUB // _R
_ZR = 80
_ZCHUNKS = _S // _ZR
_CD = _D // 2


def _sc_body(x_hbm, gi_hbm, t_hbm, out_hbm,
             x_a, x_b, vals_a, vals_b, idx_a, idx_b, idx_sa, idx_sb,
             zb_v, ob_v, t_v,
             sem_xa, sem_xb, sem_ia, sem_ib, sem_sa, sem_sb,
             acc_sh):
    c = lax.axis_index("c")
    s = lax.axis_index("s")
    col0 = c * _CD
    base = s * _ROWS_PER_SUB

    pltpu.sync_copy(t_hbm, t_v)
    tv = t_v[...]

    def zero_row(r, carry):
        for k in range(_D // 16):
            zb_v[r, pl.ds(k * 16, 16)] = jnp.zeros((16,), jnp.float32)
        return carry

    lax.fori_loop(0, _ZR, zero_row, 0)

    zlo = (s * _ZCHUNKS) // _NSUB
    zhi = ((s + 1) * _ZCHUNKS) // _NSUB

    def zero_chunk(j, carry):
        pltpu.sync_copy(zb_v, acc_sh.at[pl.ds(j * _ZR, _ZR)])
        return carry

    lax.fori_loop(zlo, zhi, zero_chunk, 0)
    plsc.subcore_barrier()

    def issue_in(t, x_buf, idx_buf, sem_x, sem_i):
        row0 = base + t * _R
        pltpu.async_copy(gi_hbm.at[pl.ds(row0, _R)], idx_buf, sem_i)
        pltpu.async_copy(x_hbm.at[pl.ds(row0, _R), pl.ds(col0, _CD)],
                         x_buf, sem_x)

    def wait_in(x_buf, idx_buf, sem_x, sem_i):
        pltpu.make_async_copy(gi_hbm.at[pl.ds(0, _R)], idx_buf, sem_i).wait()
        pltpu.make_async_copy(x_hbm.at[pl.ds(0, _R), pl.ds(0, _CD)],
                              x_buf, sem_x).wait()

    def compute(x_buf, vals_buf):
        def row_body(r, carry):
            for k in range(_CD // 16):
                xv = x_buf[r, pl.ds(k * 16, 16)]
                e = jnp.exp(xv * tv)
                vals_buf[r, pl.ds(_CD + k * 16, 16)] = e
                vals_buf[r, pl.ds(k * 16, 16)] = xv * e
            return carry

        lax.fori_loop(0, _R, row_body, 0)

    def issue_scatter(vals_buf, idx_buf, idx_s_buf, sem_s):
        for k in range(_R // 16):
            idx_s_buf[pl.ds(k * 16, 16)] = idx_buf[pl.ds(k * 16, 16)]
        pltpu.async_copy(vals_buf, acc_sh.at[idx_s_buf], sem_s, add=True)

    def wait_scatter(vals_buf, idx_s_buf, sem_s):
        pltpu.make_async_copy(vals_buf, acc_sh.at[idx_s_buf], sem_s).wait()

    issue_in(0, x_a, idx_a, sem_xa, sem_ia)

    @pl.loop(0, _TCH, step=2)
    def chunk_pair(t):
        issue_in(t + 1, x_b, idx_b, sem_xb, sem_ib)
        wait_in(x_a, idx_a, sem_xa, sem_ia)

        @pl.when(t >= 2)
        def _():
            wait_scatter(vals_a, idx_sa, sem_sa)

        compute(x_a, vals_a)
        issue_scatter(vals_a, idx_a, idx_sa, sem_sa)

        @pl.when(t < _TCH - 2)
        def _():
            issue_in(t + 2, x_a, idx_a, sem_xa, sem_ia)

        wait_in(x_b, idx_b, sem_xb, sem_ib)

        @pl.when(t >= 2)
        def _():
            wait_scatter(vals_b, idx_sb, sem_sb)

        compute(x_b, vals_b)
        issue_scatter(vals_b, idx_b, idx_sb, sem_sb)

    wait_scatter(vals_a, idx_sa, sem_sa)
    wait_scatter(vals_b, idx_sb, sem_sb)
    plsc.subcore_barrier()

    def div_chunk(j, carry):
        r0 = j * _ZR
        pltpu.sync_copy(acc_sh.at[pl.ds(r0, _ZR)], zb_v)

        def div_row(r, carry2):
            for k in range(_CD // 16):
                nu = zb_v[r, pl.ds(k * 16, 16)]
                de = zb_v[r, pl.ds(_CD + k * 16, 16)]
                q = nu / de
                ob_v[r, pl.ds(k * 16, 16)] = jnp.where(
                    de != 0.0, q, jnp.zeros((16,), jnp.float32))
            return carry2

        lax.fori_loop(0, _ZR, div_row, 0)
        pltpu.sync_copy(ob_v, out_hbm.at[pl.ds(r0, _ZR), pl.ds(col0, _CD)])
        return carry

    lax.fori_loop(zlo, zhi, div_chunk, 0)


_sc_pool = functools.partial(
    pl.kernel,
    mesh=plsc.VectorSubcoreMesh(core_axis_name="c", subcore_axis_name="s"),
    out_type=jax.ShapeDtypeStruct((_S, _D), jnp.float32),
    compiler_params=pltpu.CompilerParams(use_tc_tiling_on_sc=False),
    scratch_types=[
        pltpu.VMEM((_R, _CD), jnp.float32),
        pltpu.VMEM((_R, _CD), jnp.float32),
        pltpu.VMEM((_R, _D), jnp.float32),
        pltpu.VMEM((_R, _D), jnp.float32),
        pltpu.VMEM((_R,), jnp.int32),
        pltpu.VMEM((_R,), jnp.int32),
        pltpu.VMEM((_R,), jnp.int32),
        pltpu.VMEM((_R,), jnp.int32),
        pltpu.VMEM((_ZR, _D), jnp.float32),
        pltpu.VMEM((_ZR, _CD), jnp.float32),
        pltpu.VMEM((16,), jnp.float32),
        pltpu.SemaphoreType.DMA,
        pltpu.SemaphoreType.DMA,
        pltpu.SemaphoreType.DMA,
        pltpu.SemaphoreType.DMA,
        pltpu.SemaphoreType.DMA,
        pltpu.SemaphoreType.DMA,
        pltpu.VMEM_SHARED((_S, _D), jnp.float32),
    ],
)(_sc_body)


def kernel(x, dim, group_indices, temperature):
    gi = group_indices.astype(jnp.int32)
    t16 = jnp.broadcast_to(
        temperature.reshape(-1)[:1].astype(jnp.float32), (16,))
    pooled = _sc_pool(x, gi, t16)
    return jnp.where(dim == 0, pooled, jnp.zeros_like(pooled))


# speedup vs baseline: 10.9673x; 4.5367x over previous
import functools

import jax
import jax.numpy as jnp
from jax import lax
from jax.experimental import pallas as pl
from jax.experimental.pallas import tpu as pltpu
from jax.experimental.pallas import tpu_sc as plsc

_N = 320000
_D = 128
_S = 10000

_NSUB = 16
_ROWS_PER_SUB = _N // _NSUB
_R = 80
_TCH = _ROWS_PER_SUB // _R
_ZR = 80
_ZCHUNKS = _S // _ZR
_CD = _D // 2


def _sc_body(x_hbm, gi_hbm, t_hbm, out_hbm,
             x_a, x_b, vals_a, vals_b, idx_a, idx_b, idx_sa, idx_sb,
             zb_v, ob_v, t_v,
             sem_xa, sem_xb, sem_ia, sem_ib, sem_sa, sem_sb,
             acc_sh):
    c = lax.axis_index("c")
    s = lax.axis_index("s")
    col0 = c * _CD
    base = s * _ROWS_PER_SUB

    pltpu.sync_copy(t_hbm, t_v)
    tv = t_v[...]

    @plsc.parallel_loop(0, _ZR, unroll=4)
    def zero_row(r):
        for k in range(_D // 16):
            zb_v[r, pl.ds(k * 16, 16)] = jnp.zeros((16,), jnp.float32)

    zlo = (s * _ZCHUNKS) // _NSUB
    zhi = ((s + 1) * _ZCHUNKS) // _NSUB

    def zero_chunk(j, carry):
        pltpu.sync_copy(zb_v, acc_sh.at[pl.ds(j * _ZR, _ZR)])
        return carry

    lax.fori_loop(zlo, zhi, zero_chunk, 0)
    plsc.subcore_barrier()

    def issue_in(t, x_buf, idx_buf, sem_x, sem_i):
        row0 = base + t * _R
        pltpu.async_copy(gi_hbm.at[pl.ds(row0, _R)], idx_buf, sem_i)
        pltpu.async_copy(x_hbm.at[pl.ds(row0, _R), pl.ds(col0, _CD)],
                         x_buf, sem_x)

    def wait_in(x_buf, idx_buf, sem_x, sem_i):
        pltpu.make_async_copy(gi_hbm.at[pl.ds(0, _R)], idx_buf, sem_i).wait()
        pltpu.make_async_copy(x_hbm.at[pl.ds(0, _R), pl.ds(0, _CD)],
                              x_buf, sem_x).wait()

    def compute(x_buf, vals_buf):
        @plsc.parallel_loop(0, _R, unroll=4)
        def row_body(r):
            xs = [x_buf[r, pl.ds(k * 16, 16)] for k in range(_CD // 16)]
            es = [jnp.exp(xv * tv) for xv in xs]
            for k in range(_CD // 16):
                vals_buf[r, pl.ds(_CD + k * 16, 16)] = es[k]
            for k in range(_CD // 16):
                vals_buf[r, pl.ds(k * 16, 16)] = xs[k] * es[k]

    def issue_scatter(vals_buf, idx_buf, idx_s_buf, sem_s):
        for k in range(_R // 16):
            idx_s_buf[pl.ds(k * 16, 16)] = idx_buf[pl.ds(k * 16, 16)]
        pltpu.async_copy(vals_buf, acc_sh.at[idx_s_buf], sem_s, add=True)

    def wait_scatter(vals_buf, idx_s_buf, sem_s):
        pltpu.make_async_copy(vals_buf, acc_sh.at[idx_s_buf], sem_s).wait()

    issue_in(0, x_a, idx_a, sem_xa, sem_ia)

    @pl.loop(0, _TCH, step=2)
    def chunk_pair(t):
        issue_in(t + 1, x_b, idx_b, sem_xb, sem_ib)
        wait_in(x_a, idx_a, sem_xa, sem_ia)

        @pl.when(t >= 2)
        def _():
            wait_scatter(vals_a, idx_sa, sem_sa)

        compute(x_a, vals_a)
        issue_scatter(vals_a, idx_a, idx_sa, sem_sa)

        @pl.when(t < _TCH - 2)
        def _():
            issue_in(t + 2, x_a, idx_a, sem_xa, sem_ia)

        wait_in(x_b, idx_b, sem_xb, sem_ib)

        @pl.when(t >= 2)
        def _():
            wait_scatter(vals_b, idx_sb, sem_sb)

        compute(x_b, vals_b)
        issue_scatter(vals_b, idx_b, idx_sb, sem_sb)

    wait_scatter(vals_a, idx_sa, sem_sa)
    wait_scatter(vals_b, idx_sb, sem_sb)
    plsc.subcore_barrier()

    def div_chunk(j, carry):
        r0 = j * _ZR
        pltpu.sync_copy(acc_sh.at[pl.ds(r0, _ZR)], zb_v)

        @plsc.parallel_loop(0, _ZR, unroll=4)
        def div_row(r):
            nus = [zb_v[r, pl.ds(k * 16, 16)] for k in range(_CD // 16)]
            des = [zb_v[r, pl.ds(_CD + k * 16, 16)] for k in range(_CD // 16)]
            qs = [jnp.where(de != 0.0, nu / de, jnp.zeros((16,), jnp.float32))
                  for nu, de in zip(nus, des)]
            for k in range(_CD // 16):
                ob_v[r, pl.ds(k * 16, 16)] = qs[k]
        pltpu.sync_copy(ob_v, out_hbm.at[pl.ds(r0, _ZR), pl.ds(col0, _CD)])
        return carry

    lax.fori_loop(zlo, zhi, div_chunk, 0)


_sc_pool = functools.partial(
    pl.kernel,
    mesh=plsc.VectorSubcoreMesh(core_axis_name="c", subcore_axis_name="s"),
    out_type=jax.ShapeDtypeStruct((_S, _D), jnp.float32),
    compiler_params=pltpu.CompilerParams(use_tc_tiling_on_sc=False),
    scratch_types=[
        pltpu.VMEM((_R, _CD), jnp.float32),
        pltpu.VMEM((_R, _CD), jnp.float32),
        pltpu.VMEM((_R, _D), jnp.float32),
        pltpu.VMEM((_R, _D), jnp.float32),
        pltpu.VMEM((_R,), jnp.int32),
        pltpu.VMEM((_R,), jnp.int32),
        pltpu.VMEM((_R,), jnp.int32),
        pltpu.VMEM((_R,), jnp.int32),
        pltpu.VMEM((_ZR, _D), jnp.float32),
        pltpu.VMEM((_ZR, _CD), jnp.float32),
        pltpu.VMEM((16,), jnp.float32),
        pltpu.SemaphoreType.DMA,
        pltpu.SemaphoreType.DMA,
        pltpu.SemaphoreType.DMA,
        pltpu.SemaphoreType.DMA,
        pltpu.SemaphoreType.DMA,
        pltpu.SemaphoreType.DMA,
        pltpu.VMEM_SHARED((_S, _D), jnp.float32),
    ],
)(_sc_body)


def kernel(x, dim, group_indices, temperature):
    gi = group_indices.astype(jnp.int32)
    t16 = jnp.broadcast_to(
        temperature.reshape(-1)[:1].astype(jnp.float32), (16,))
    pooled = _sc_pool(x, gi, t16)
    return jnp.where(dim == 0, pooled, jnp.zeros_like(pooled))
